# Initial kernel scaffold; baseline (speedup 1.0000x reference)
import jax
import jax.numpy as jnp
from jax.experimental import pallas as pl


def kernel(windows, centers, center_emb, output_embs, noises):
    raise NotImplementedError("write your pallas kernel here")


# speedup vs baseline: 1.5286x; 1.5286x over previous
import functools

import jax
import jax.numpy as jnp
from jax import lax
from jax.experimental import pallas as pl
from jax.experimental.pallas import tpu as pltpu
from jax.experimental.pallas import tpu_sc as plsc

_V = 100000
_D = 64
_W = 4
_NS = 10
_LANES = 16
_NWORK = 32


def _sc_scores(windows_t, centers, center_emb, emb_flat, noises_flat, batch):
    bpw = batch // _NWORK
    rows_per_pos = bpw * (1 + _NS)
    nchunks = 1 + _NS

    mesh = plsc.VectorSubcoreMesh(core_axis_name="c", subcore_axis_name="s")
    info = plsc.get_sparse_core_info()
    nc = info.num_cores

    @functools.partial(
        pl.kernel,
        mesh=mesh,
        out_type=jax.ShapeDtypeStruct((_W, _NWORK, bpw, _LANES), jnp.float32),
        compiler_params=pltpu.CompilerParams(
            needs_layout_passes=False, use_tc_tiling_on_sc=False),
        scratch_types=[
            pltpu.VMEM((bpw,), jnp.int32),
            pltpu.VMEM((bpw,), jnp.int32),
            pltpu.VMEM((bpw * _NS,), jnp.int32),
            pltpu.VMEM((bpw, _D), jnp.float32),
            pltpu.VMEM((rows_per_pos, _D), jnp.float32),
            pltpu.VMEM((bpw, _LANES), jnp.float32),
            pltpu.SemaphoreType.DMA,
            pltpu.SemaphoreType.DMA,
        ],
    )
    def body(win_hbm, cen_hbm, cemb_hbm, oemb_hbm, noise_hbm, out_hbm,
             cidx_v, widx_v, nidx_v, ctx_v, rows_v, sc_v, sem_ctx, sem_rows):
        wid = lax.axis_index("s") * nc + lax.axis_index("c")
        base = wid * bpw

        pltpu.sync_copy(cen_hbm.at[pl.ds(base, bpw)], cidx_v)
        ctx_cp = pltpu.async_copy(cemb_hbm.at[cidx_v], ctx_v, sem_ctx)
        ctx_cp.wait()

        for pos in range(_W):
            pltpu.sync_copy(win_hbm.at[pos, pl.ds(base, bpw)], widx_v)
            pltpu.sync_copy(
                noise_hbm.at[pos, pl.ds(base * _NS, bpw * _NS)], nidx_v)
            off = jnp.int32(pos * _V)
            for i in range(bpw // _LANES):
                sl = pl.ds(i * _LANES, _LANES)
                widx_v[sl] = widx_v[sl] + off
            for i in range(bpw * _NS // _LANES):
                sl = pl.ds(i * _LANES, _LANES)
                nidx_v[sl] = nidx_v[sl] + off

            cps = [pltpu.async_copy(
                oemb_hbm.at[widx_v], rows_v.at[pl.ds(0, bpw)], sem_rows)]
            for c in range(1, nchunks):
                idx = nidx_v.at[pl.ds((c - 1) * bpw, bpw)]
                dst = rows_v.at[pl.ds(c * bpw, bpw)]
                cps.append(pltpu.async_copy(oemb_hbm.at[idx], dst, sem_rows))
            for cp in cps:
                cp.wait()

            lane = lax.iota(jnp.int32, _LANES)

            def dot_loop(b, carry):
                cvs = [ctx_v[b, pl.ds(k * _LANES, _LANES)] for k in range(_D // _LANES)]

                def row_dot(r):
                    acc = rows_v[r, pl.ds(0, _LANES)] * cvs[0]
                    for k in range(1, _D // _LANES):
                        acc = acc + rows_v[r, pl.ds(k * _LANES, _LANES)] * cvs[k]
                    return jnp.sum(acc)

                vec = jnp.full((_LANES,), 30.0, jnp.float32)
                vec = jnp.where(lane == 0, row_dot(b), vec)
                for n in range(_NS):
                    j = bpw + b * _NS + n
                    vec = jnp.where(lane == n + 1, row_dot(j), vec)
                sc_v[b, :] = vec
                return carry

            lax.fori_loop(0, bpw, dot_loop, jnp.int32(0))
            pltpu.sync_copy(sc_v, out_hbm.at[pos, wid])

    return body(windows_t, centers, center_emb, emb_flat, noises_flat)


def _tc_loss(scores2d):
    def body(s_ref, o_ref):
        x = s_ref[...]
        sub = lax.broadcasted_iota(jnp.int32, x.shape, 1) % _LANES
        x = jnp.where((sub >= 1) & (sub <= _NS), -x, x)
        o_ref[...] = jnp.broadcast_to(jnp.sum(jax.nn.softplus(-x)), (1, 1))

    return pl.pallas_call(
        body,
        out_shape=jax.ShapeDtypeStruct((1, 1), jnp.float32),
    )(scores2d)


def kernel(windows, centers, center_emb, output_embs, noises):
    batch = windows.shape[0]
    bpw = batch // _NWORK
    windows_t = windows.T.astype(jnp.int32)
    noises_flat = noises.reshape(_W, batch * _NS)
    emb_flat = output_embs.reshape(_W * _V, _D)
    scores = _sc_scores(windows_t, centers.astype(jnp.int32), center_emb,
                        emb_flat, noises_flat, batch)
    scores2d = scores.reshape(_W * _NWORK * bpw * _LANES // 128, 128)
    total = _tc_loss(scores2d)
    return (total[0, 0], jnp.int32(windows.size))
